# Initial kernel scaffold; baseline (speedup 1.0000x reference)
import jax
import jax.numpy as jnp
from jax.experimental import pallas as pl


def kernel(x, edge_index_beam, edge_index_column, W_beam, b_beam, W_column, b_column):
    raise NotImplementedError("write your pallas kernel here")


# speedup vs baseline: 4.8848x; 4.8848x over previous
import dataclasses
import functools

import jax
import jax.numpy as jnp
from jax import lax
from jax.experimental import pallas as pl
from jax.experimental.pallas import tpu as pltpu
from jax.experimental.pallas import tpu_sc as plsc

N = 50000
D = 128
E = 400000

N_PAD = 50176
E_PAD = 409600
E_SUB = E_PAD // 16
CHUNK = 3200
COLS_PER_SUB = D // 16

TILE = 3584
GRID = N_PAD // TILE


def _sc_mesh():
    return plsc.VectorSubcoreMesh(core_axis_name="c", subcore_axis_name="s")


def _sc_params():
    cp = pltpu.CompilerParams()
    if "needs_layout_passes" in pltpu.CompilerParams.__dataclass_fields__:
        cp = dataclasses.replace(cp, needs_layout_passes=False)
    return cp


def _histogram(edges):
    @functools.partial(
        pl.kernel,
        out_type=jax.ShapeDtypeStruct((32, N_PAD), jnp.float32),
        mesh=_sc_mesh(),
        compiler_params=_sc_params(),
        scratch_types=[
            pltpu.VMEM((N_PAD,), jnp.float32),
            pltpu.VMEM((CHUNK,), jnp.int32),
        ],
    )
    def hist_kernel(e_hbm, out_hbm, hist, dstv):
        c = lax.axis_index("c")
        s = lax.axis_index("s")
        zeros16 = jnp.zeros((16,), jnp.float32)
        ones16 = jnp.ones((16,), jnp.float32)

        @pl.loop(0, N_PAD, step=16)
        def _(i):
            hist[pl.ds(i, 16)] = zeros16

        base = s * E_SUB
        dst_row = 2 * c + 1

        @pl.loop(0, E_SUB, step=CHUNK)
        def _(off):
            pltpu.sync_copy(e_hbm.at[dst_row, pl.ds(base + off, CHUNK)], dstv)

            @pl.loop(0, CHUNK, step=16)
            def _(i):
                dv = dstv[pl.ds(i, 16)]
                plsc.addupdate_scatter(hist, [dv], ones16)

        pltpu.sync_copy(hist, out_hbm.at[c * 16 + s])

    return hist_kernel(edges)


def _matmul_body(x_ref, w_ref, out_ref):
    xt = x_ref[...].T
    out_ref[0:128, :] = jnp.dot(w_ref[0], xt, preferred_element_type=jnp.float32)
    out_ref[128:256, :] = jnp.dot(w_ref[1], xt, preferred_element_type=jnp.float32)


def _matmul(xp, w_stack):
    return pl.pallas_call(
        _matmul_body,
        grid=(GRID,),
        in_specs=[
            pl.BlockSpec((TILE, 128), lambda i: (i, 0)),
            pl.BlockSpec((2, 128, 128), lambda i: (0, 0, 0)),
        ],
        out_specs=pl.BlockSpec((256, TILE), lambda i: (0, i)),
        out_shape=jax.ShapeDtypeStruct((256, N_PAD), jnp.float32),
    )(xp, w_stack)


def _scale_body(part_ref, h_ref, g_ref, dinv_ref):
    deg_b = jnp.sum(part_ref[0:16, :], axis=0, keepdims=True) + 1.0
    deg_c = jnp.sum(part_ref[16:32, :], axis=0, keepdims=True) + 1.0
    dinv_b = lax.rsqrt(deg_b)
    dinv_c = lax.rsqrt(deg_c)
    dinv_ref[0:1, :] = dinv_b
    dinv_ref[1:2, :] = dinv_c
    g_ref[0:128, :] = h_ref[0:128, :] * dinv_b
    g_ref[128:256, :] = h_ref[128:256, :] * dinv_c


def _scale(partials, hT):
    return pl.pallas_call(
        _scale_body,
        grid=(GRID,),
        in_specs=[
            pl.BlockSpec((32, TILE), lambda i: (0, i)),
            pl.BlockSpec((256, TILE), lambda i: (0, i)),
        ],
        out_specs=[
            pl.BlockSpec((256, TILE), lambda i: (0, i)),
            pl.BlockSpec((2, TILE), lambda i: (0, i)),
        ],
        out_shape=[
            jax.ShapeDtypeStruct((256, N_PAD), jnp.float32),
            jax.ShapeDtypeStruct((2, N_PAD), jnp.float32),
        ],
    )(partials, hT)


def _aggregate(gT, edges):
    @functools.partial(
        pl.kernel,
        out_type=jax.ShapeDtypeStruct((256, N_PAD), jnp.float32),
        mesh=_sc_mesh(),
        compiler_params=_sc_params(),
        scratch_types=[
            pltpu.VMEM((N_PAD,), jnp.float32),
            pltpu.VMEM((N_PAD,), jnp.float32),
            pltpu.VMEM((CHUNK,), jnp.int32),
            pltpu.VMEM((CHUNK,), jnp.int32),
        ],
    )
    def agg_kernel(g_hbm, e_hbm, out_hbm, gcol, acc, srcv, dstv):
        c = lax.axis_index("c")
        s = lax.axis_index("s")
        zeros16 = jnp.zeros((16,), jnp.float32)
        src_row = 2 * c

        for k in range(COLS_PER_SUB):
            row = c * 128 + s * COLS_PER_SUB + k
            pltpu.sync_copy(g_hbm.at[row], gcol)

            @pl.loop(0, N_PAD, step=16)
            def _(i):
                acc[pl.ds(i, 16)] = zeros16

            @pl.loop(0, E_PAD, step=CHUNK)
            def _(off):
                pltpu.sync_copy(e_hbm.at[src_row, pl.ds(off, CHUNK)], srcv)
                pltpu.sync_copy(e_hbm.at[src_row + 1, pl.ds(off, CHUNK)], dstv)

                @pl.loop(0, CHUNK, step=16)
                def _(i):
                    sv = srcv[pl.ds(i, 16)]
                    dv = dstv[pl.ds(i, 16)]
                    vals = plsc.load_gather(gcol, [sv])
                    plsc.addupdate_scatter(acc, [dv], vals)

            pltpu.sync_copy(acc, out_hbm.at[row])

    return agg_kernel(gT, edges)


def _final_body(s_ref, g_ref, dinv_ref, b_ref, out_ref):
    tmp_b = (s_ref[0:128, :] + g_ref[0:128, :]) * dinv_ref[0:1, :]
    tmp_c = (s_ref[128:256, :] + g_ref[128:256, :]) * dinv_ref[1:2, :]
    bias = (b_ref[0] + b_ref[1])[None, :]
    out_ref[...] = (tmp_b + tmp_c).T + bias


def _final(sT, gT, dinv, b_stack):
    return pl.pallas_call(
        _final_body,
        grid=(GRID,),
        in_specs=[
            pl.BlockSpec((256, TILE), lambda i: (0, i)),
            pl.BlockSpec((256, TILE), lambda i: (0, i)),
            pl.BlockSpec((2, TILE), lambda i: (0, i)),
            pl.BlockSpec((2, 128), lambda i: (0, 0)),
        ],
        out_specs=pl.BlockSpec((TILE, 128), lambda i: (i, 0)),
        out_shape=jax.ShapeDtypeStruct((N_PAD, 128), jnp.float32),
    )(sT, gT, dinv, b_stack)


def kernel(x, edge_index_beam, edge_index_column, W_beam, b_beam, W_column, b_column):
    pad_n = E_PAD - E
    rows = []
    for ei in (edge_index_beam, edge_index_column):
        ei = ei.astype(jnp.int32)
        rows.append(jnp.concatenate([ei[0], jnp.zeros((pad_n,), jnp.int32)]))
        rows.append(jnp.concatenate([ei[1], jnp.full((pad_n,), N_PAD - 1, jnp.int32)]))
    edges = jnp.stack(rows)

    xp = jnp.pad(x, ((0, N_PAD - N), (0, 0)))
    w_stack = jnp.stack([W_beam, W_column])
    b_stack = jnp.stack([b_beam, b_column])

    partials = _histogram(edges)
    hT = _matmul(xp, w_stack)
    gT, dinv = _scale(partials, hT)
    sT = _aggregate(gT, edges)
    out_p = _final(sT, gT, dinv, b_stack)

    return out_p[:N]
